# baseline (speedup 1.0000x reference)
import functools

import jax
import jax.numpy as jnp
from jax import lax
from jax.experimental import pallas as pl
from jax.experimental.pallas import tpu as pltpu
from jax.experimental.pallas import tpu_sc as plsc

VOCAB = 30522
HIDDEN = 768
BATCH = 64
SEQ = 512
EPS = 1e-12

NC = 2
NS = 16
NW = NC * NS
PPW = SEQ // NW
NJ = HIDDEN // 16


def _lanesum16(v):
    idx = lax.iota(jnp.int32, 16)
    dnums = lax.GatherDimensionNumbers(
        offset_dims=(), collapsed_slice_dims=(0,), start_index_map=(0,))
    for sh in (8, 4, 2, 1):
        perm = (idx ^ sh)[:, None]
        v = v + lax.gather(v, perm, dimension_numbers=dnums, slice_sizes=(1,),
                           unique_indices=True,
                           mode=lax.GatherScatterMode.PROMISE_IN_BOUNDS)
    return v


def _rsqrt16(v):
    x2 = v * 0.5
    i = lax.bitcast_convert_type(v, jnp.int32)
    i = jnp.int32(0x5F3759DF) - (i >> 1)
    y = lax.bitcast_convert_type(i, jnp.float32)
    y = y * (1.5 - x2 * y * y)
    y = y * (1.5 - x2 * y * y)
    y = y * (1.5 - x2 * y * y)
    y = y * (1.5 - x2 * y * y)
    return y


def _sc_body(ids_w, wemb, pemb, gamma, beta, out,
             idx_v, pos_v, g_v, b_v,
             in_a, in_b, out_a, out_b,
             gsem_a, gsem_b, ssem_a, ssem_b):
    c = lax.axis_index("c")
    s = lax.axis_index("s")
    w = s * NC + c

    pltpu.sync_copy(ids_w.at[w], idx_v)
    pltpu.sync_copy(pemb.at[pl.ds(w * PPW, PPW)], pos_v)
    pltpu.sync_copy(gamma, g_v)
    pltpu.sync_copy(beta, b_v)

    def gather(b, buf, sem):
        pltpu.async_copy(wemb.at[idx_v.at[pl.ds(b * PPW, PPW)]], buf, sem)

    RG = 4

    def compute(xin, xout):
        def group_body(g, carry2):
            r0 = g * RG
            means = []
            invs = []
            for rr in range(RG):
                r = r0 + rr
                acc = [jnp.zeros((16,), jnp.float32) for _ in range(4)]
                accq = [jnp.zeros((16,), jnp.float32) for _ in range(4)]
                for j in range(NJ):
                    sl = pl.ds(j * 16, 16)
                    x = xin[r, sl] + pos_v[r, sl]
                    xout[r, sl] = x
                    acc[j % 4] = acc[j % 4] + x
                    accq[j % 4] = accq[j % 4] + x * x
                sum_v = (acc[0] + acc[1]) + (acc[2] + acc[3])
                sq_v = (accq[0] + accq[1]) + (accq[2] + accq[3])
                mean_v = _lanesum16(sum_v) * (1.0 / HIDDEN)
                msq_v = _lanesum16(sq_v) * (1.0 / HIDDEN)
                var_v = msq_v - mean_v * mean_v
                means.append(mean_v)
                invs.append(_rsqrt16(var_v + EPS))
            for j in range(NJ):
                sl = pl.ds(j * 16, 16)
                gv = g_v[sl]
                bv = b_v[sl]
                for rr in range(RG):
                    r = r0 + rr
                    xout[r, sl] = (xout[r, sl] - means[rr]) * invs[rr] * gv + bv
            return carry2

        lax.fori_loop(0, PPW // RG, group_body, 0)

    def halfstep(i, b, xin, xout, gsem, ssem):
        @pl.when(i > 0)
        def _():
            pltpu.make_async_copy(xout, out.at[0, pl.ds(w * PPW, PPW)], ssem).wait()

        pltpu.make_async_copy(wemb.at[pl.ds(0, PPW)], xin, gsem).wait()
        compute(xin, xout)

        @pl.when(b + 2 < BATCH)
        def _():
            gather(b + 2, xin, gsem)

        pltpu.async_copy(xout, out.at[b, pl.ds(w * PPW, PPW)], ssem)

    gather(0, in_a, gsem_a)
    gather(1, in_b, gsem_b)

    def loop_body(i, carry):
        halfstep(i, 2 * i, in_a, out_a, gsem_a, ssem_a)
        halfstep(i, 2 * i + 1, in_b, out_b, gsem_b, ssem_b)
        return carry

    lax.fori_loop(0, BATCH // 2, loop_body, 0)

    pltpu.make_async_copy(out_a, out.at[0, pl.ds(w * PPW, PPW)], ssem_a).wait()
    pltpu.make_async_copy(out_b, out.at[0, pl.ds(w * PPW, PPW)], ssem_b).wait()


@functools.partial(jax.jit, static_argnames=())
def _run(ids_w, word_emb, pos_emb, ln_gamma, ln_beta):
    kern = pl.kernel(
        _sc_body,
        out_type=jax.ShapeDtypeStruct((BATCH, SEQ, HIDDEN), jnp.float32),
        mesh=plsc.VectorSubcoreMesh(core_axis_name="c", subcore_axis_name="s"),
        scratch_types=[
            pltpu.VMEM((BATCH * PPW,), jnp.int32),
            pltpu.VMEM((PPW, HIDDEN), jnp.float32),
            pltpu.VMEM((HIDDEN,), jnp.float32),
            pltpu.VMEM((HIDDEN,), jnp.float32),
            pltpu.VMEM((PPW, HIDDEN), jnp.float32),
            pltpu.VMEM((PPW, HIDDEN), jnp.float32),
            pltpu.VMEM((PPW, HIDDEN), jnp.float32),
            pltpu.VMEM((PPW, HIDDEN), jnp.float32),
            pltpu.SemaphoreType.DMA,
            pltpu.SemaphoreType.DMA,
            pltpu.SemaphoreType.DMA,
            pltpu.SemaphoreType.DMA,
        ],
    )
    return kern(ids_w, word_emb, pos_emb, ln_gamma, ln_beta)


def kernel(input_ids, word_emb, pos_emb, ln_gamma, ln_beta):
    ids_w = (
        input_ids.T.reshape(NW, PPW, BATCH)
        .transpose(0, 2, 1)
        .reshape(NW, BATCH * PPW)
    )
    return _run(ids_w, word_emb, pos_emb, ln_gamma, ln_beta)


# speedup vs baseline: 3.2842x; 3.2842x over previous
import functools

import jax
import jax.numpy as jnp
from jax import lax
from jax.experimental import pallas as pl
from jax.experimental.pallas import tpu as pltpu
from jax.experimental.pallas import tpu_sc as plsc

VOCAB = 30522
HIDDEN = 768
BATCH = 64
SEQ = 512
EPS = 1e-12

NC = 2
NS = 16
NW = NC * NS
PPW = SEQ // NW
NJ = HIDDEN // 16


def _lanesum16(v):
    idx = lax.iota(jnp.int32, 16)
    dnums = lax.GatherDimensionNumbers(
        offset_dims=(), collapsed_slice_dims=(0,), start_index_map=(0,))
    for sh in (8, 4, 2, 1):
        perm = (idx ^ sh)[:, None]
        v = v + lax.gather(v, perm, dimension_numbers=dnums, slice_sizes=(1,),
                           unique_indices=True,
                           mode=lax.GatherScatterMode.PROMISE_IN_BOUNDS)
    return v


def _rsqrt16(v):
    x2 = v * 0.5
    i = lax.bitcast_convert_type(v, jnp.int32)
    i = jnp.int32(0x5F3759DF) - (i >> 1)
    y = lax.bitcast_convert_type(i, jnp.float32)
    y = y * (1.5 - x2 * y * y)
    y = y * (1.5 - x2 * y * y)
    y = y * (1.5 - x2 * y * y)
    y = y * (1.5 - x2 * y * y)
    return y


def _sc_body(ids_w, wemb, pemb, gamma, beta, out,
             idx_v, pos_v, g_v, b_v,
             in_a, in_b, out_a, out_b,
             gsem_a, gsem_b, ssem_a, ssem_b):
    c = lax.axis_index("c")
    s = lax.axis_index("s")
    w = s * NC + c

    pltpu.sync_copy(ids_w.at[w], idx_v)
    pltpu.sync_copy(pemb.at[pl.ds(w * PPW, PPW)], pos_v)
    pltpu.sync_copy(gamma, g_v)
    pltpu.sync_copy(beta, b_v)

    def gather(b, buf, sem):
        pltpu.async_copy(wemb.at[idx_v.at[pl.ds(b * PPW, PPW)]], buf, sem)

    def compute(xin, xout):
        @plsc.parallel_loop(0, PPW, 1, unroll=2)
        def row_body(r):
            sum_v = jnp.zeros((16,), jnp.float32)
            sq_v = jnp.zeros((16,), jnp.float32)
            for j in range(NJ):
                sl = pl.ds(j * 16, 16)
                x = xin[r, sl] + pos_v[r, sl]
                xout[r, sl] = x
                sum_v = sum_v + x
                sq_v = sq_v + x * x
            mean_v = _lanesum16(sum_v) * (1.0 / HIDDEN)
            msq_v = _lanesum16(sq_v) * (1.0 / HIDDEN)
            var_v = msq_v - mean_v * mean_v
            inv_v = _rsqrt16(var_v + EPS)
            for j in range(NJ):
                sl = pl.ds(j * 16, 16)
                xout[r, sl] = (xout[r, sl] - mean_v) * inv_v * g_v[sl] + b_v[sl]

    def halfstep(i, b, xin, xout, gsem, ssem):
        @pl.when(i > 0)
        def _():
            pltpu.make_async_copy(xout, out.at[0, pl.ds(w * PPW, PPW)], ssem).wait()

        pltpu.make_async_copy(wemb.at[pl.ds(0, PPW)], xin, gsem).wait()
        compute(xin, xout)

        @pl.when(b + 2 < BATCH)
        def _():
            gather(b + 2, xin, gsem)

        pltpu.async_copy(xout, out.at[b, pl.ds(w * PPW, PPW)], ssem)

    gather(0, in_a, gsem_a)
    gather(1, in_b, gsem_b)

    def loop_body(i, carry):
        halfstep(i, 2 * i, in_a, out_a, gsem_a, ssem_a)
        halfstep(i, 2 * i + 1, in_b, out_b, gsem_b, ssem_b)
        return carry

    lax.fori_loop(0, BATCH // 2, loop_body, 0)

    pltpu.make_async_copy(out_a, out.at[0, pl.ds(w * PPW, PPW)], ssem_a).wait()
    pltpu.make_async_copy(out_b, out.at[0, pl.ds(w * PPW, PPW)], ssem_b).wait()


@functools.partial(jax.jit, static_argnames=())
def _run(ids_w, word_emb, pos_emb, ln_gamma, ln_beta):
    kern = pl.kernel(
        _sc_body,
        out_type=jax.ShapeDtypeStruct((BATCH, SEQ, HIDDEN), jnp.float32),
        mesh=plsc.VectorSubcoreMesh(core_axis_name="c", subcore_axis_name="s"),
        scratch_types=[
            pltpu.VMEM((BATCH * PPW,), jnp.int32),
            pltpu.VMEM((PPW, HIDDEN), jnp.float32),
            pltpu.VMEM((HIDDEN,), jnp.float32),
            pltpu.VMEM((HIDDEN,), jnp.float32),
            pltpu.VMEM((PPW, HIDDEN), jnp.float32),
            pltpu.VMEM((PPW, HIDDEN), jnp.float32),
            pltpu.VMEM((PPW, HIDDEN), jnp.float32),
            pltpu.VMEM((PPW, HIDDEN), jnp.float32),
            pltpu.SemaphoreType.DMA,
            pltpu.SemaphoreType.DMA,
            pltpu.SemaphoreType.DMA,
            pltpu.SemaphoreType.DMA,
        ],
    )
    return kern(ids_w, word_emb, pos_emb, ln_gamma, ln_beta)


def kernel(input_ids, word_emb, pos_emb, ln_gamma, ln_beta):
    ids_w = (
        input_ids.T.reshape(NW, PPW, BATCH)
        .transpose(0, 2, 1)
        .reshape(NW, BATCH * PPW)
    )
    return _run(ids_w, word_emb, pos_emb, ln_gamma, ln_beta)
